# Initial kernel scaffold; baseline (speedup 1.0000x reference)
import jax
import jax.numpy as jnp
from jax.experimental import pallas as pl


def kernel(feat1, feat2, curr_flow):
    raise NotImplementedError("write your pallas kernel here")


# speedup vs baseline: 4.2312x; 4.2312x over previous
import functools

import jax
import jax.numpy as jnp
from jax import lax
from jax.experimental import pallas as pl
from jax.experimental.pallas import tpu as pltpu

B = 4
C = 256
H8 = 32
W8 = 64
Q = H8 * W8
R = 4

_OFFS = []
for _y in range(-R, R + 1):
    for _x in range(abs(_y) - R, R - abs(_y) + 1):
        _OFFS.append((_x, _y))
L = len(_OFFS)


def _weight_vectors(pb, scale, n, npix):
    offs = (lax.broadcasted_iota(jnp.int32, (9, 1), 0) - R).astype(jnp.float32)
    p = pb[None, :] + offs * scale
    f = jnp.floor(pb - 4.0 * scale)
    ws = jnp.clip(f, 0.0, float(n - 3)).astype(jnp.int32)
    p0 = jnp.floor(p)
    w1 = p - p0
    w0 = 1.0 - w1
    t0 = p0.astype(jnp.int32)
    t1 = t0 + 1
    v0 = ((p0 >= 0.0) & (p0 <= float(n - 1))).astype(jnp.float32)
    v1 = ((p0 + 1.0 >= 0.0) & (p0 + 1.0 <= float(n - 1))).astype(jnp.float32)
    d0 = t0 - ws[None, :]
    d1 = t1 - ws[None, :]
    w3 = []
    for d in range(3):
        w3.append(jnp.where(d0 == d, w0 * v0, 0.0) + jnp.where(d1 == d, w1 * v1, 0.0))
    return ws, jnp.stack(w3, axis=1)


def _lookup_body(f2t_ref, f1_ref, flow_ref, out_ref):
    f2t = f2t_ref[0]
    f1 = f1_ref[0]
    fy = flow_ref[0]
    fx = flow_ref[1]

    corrT = jnp.dot(f2t, f1, preferred_element_type=jnp.float32) * (1.0 / 16.0)

    qi = lax.broadcasted_iota(jnp.int32, (Q,), 0)
    jj = (qi % W8).astype(jnp.float32)
    ii = (qi // W8).astype(jnp.float32)

    vol = corrT.reshape(H8, W8, Q)
    outs = []
    for k in range(4):
        hk = H8 >> k
        wk = W8 >> k
        if k > 0:
            a = vol.reshape(hk, 2, wk, 2, Q)
            vol = (a[:, 0, :, 0] + a[:, 0, :, 1] + a[:, 1, :, 0] + a[:, 1, :, 1]) * 0.25

        sx = float((1 << k) * (wk - 1)) / 512.0
        sy = float((1 << k) * (hk - 1)) / 256.0
        pbx = (jj + fx) * (float(wk - 1) / 512.0)
        pby = (ii + fy) * (float(hk - 1) / 256.0)
        xs, wx3 = _weight_vectors(pbx, sx, wk, Q)
        ys, wy3 = _weight_vectors(pby, sy, hk, Q)

        rows = []
        for dy in range(3):
            acc = jnp.zeros((wk, Q), jnp.float32)
            for y in range(hk):
                sel = (ys == (y - dy))[None, :]
                acc = acc + jnp.where(sel, vol[y], 0.0)
            rows.append(acc)

        lxi = lax.broadcasted_iota(jnp.int32, (wk, Q), 0)
        win = []
        for dy in range(3):
            row_dy = []
            for dx in range(3):
                m = (lxi == (xs + dx)[None, :]).astype(jnp.float32)
                row_dy.append(jnp.sum(rows[dy] * m, axis=0))
            win.append(row_dy)

        t = [[None] * 9 for _ in range(3)]
        for dy in range(3):
            for xo in range(9):
                t[dy][xo] = (wx3[xo, 0] * win[dy][0]
                             + wx3[xo, 1] * win[dy][1]
                             + wx3[xo, 2] * win[dy][2])

        lvl = []
        for (xo, yo) in _OFFS:
            v = (wy3[yo + 4, 0] * t[0][xo + 4]
                 + wy3[yo + 4, 1] * t[1][xo + 4]
                 + wy3[yo + 4, 2] * t[2][xo + 4])
            lvl.append(v)
        outs.append(jnp.stack(lvl, axis=0))

    out_ref[0] = jnp.stack(outs, axis=1)


@jax.jit
def kernel(feat1, feat2, curr_flow):
    f1 = feat1.reshape(B, C, Q)
    f2t = feat2.reshape(B, C, Q).transpose(0, 2, 1)
    flow = curr_flow.reshape(2, Q)

    out = pl.pallas_call(
        _lookup_body,
        grid=(B,),
        in_specs=[
            pl.BlockSpec((1, Q, C), lambda b: (b, 0, 0)),
            pl.BlockSpec((1, C, Q), lambda b: (b, 0, 0)),
            pl.BlockSpec((2, Q), lambda b: (0, 0)),
        ],
        out_specs=pl.BlockSpec((1, L, 4, Q), lambda b: (b, 0, 0, 0)),
        out_shape=jax.ShapeDtypeStruct((B, L, 4, Q), jnp.float32),
    )(f2t, f1, flow)
    return out.reshape(B, L, 4, H8, W8)


# speedup vs baseline: 13.2136x; 3.1229x over previous
import functools

import jax
import jax.numpy as jnp
from jax import lax
from jax.experimental import pallas as pl
from jax.experimental.pallas import tpu as pltpu

B = 4
C = 256
H8 = 32
W8 = 64
Q = H8 * W8
R = 4

_OFFS = []
for _y in range(-R, R + 1):
    for _x in range(abs(_y) - R, R - abs(_y) + 1):
        _OFFS.append((_x, _y))
L = len(_OFFS)


def _weight_vectors(pb, scale, n, npix):
    offs = (lax.broadcasted_iota(jnp.int32, (9, 1), 0) - R).astype(jnp.float32)
    p = pb[None, :] + offs * scale
    f = jnp.floor(pb - 4.0 * scale)
    ws = jnp.clip(f, 0.0, float(n - 3)).astype(jnp.int32)
    p0 = jnp.floor(p)
    w1 = p - p0
    w0 = 1.0 - w1
    t0 = p0.astype(jnp.int32)
    t1 = t0 + 1
    v0 = ((p0 >= 0.0) & (p0 <= float(n - 1))).astype(jnp.float32)
    v1 = ((p0 + 1.0 >= 0.0) & (p0 + 1.0 <= float(n - 1))).astype(jnp.float32)
    d0 = t0 - ws[None, :]
    d1 = t1 - ws[None, :]
    w3 = []
    for d in range(3):
        w3.append(jnp.where(d0 == d, w0 * v0, 0.0) + jnp.where(d1 == d, w1 * v1, 0.0))
    return ws, jnp.stack(w3, axis=1)


def _lookup_body(f2t_ref, f1_ref, flow_ref, out_ref):
    f2t = f2t_ref[0]
    f1 = f1_ref[0]
    fy = flow_ref[0]
    fx = flow_ref[1]

    corrT = jnp.dot(f2t.astype(jnp.bfloat16), f1.astype(jnp.bfloat16),
                    preferred_element_type=jnp.float32) * (1.0 / 16.0)

    qi = lax.broadcasted_iota(jnp.int32, (Q,), 0)
    jj = (qi % W8).astype(jnp.float32)
    ii = (qi // W8).astype(jnp.float32)

    vol = corrT.reshape(H8, W8, Q)
    outs = []
    for k in range(4):
        hk = H8 >> k
        wk = W8 >> k
        if k > 0:
            a = vol.reshape(hk, 2, wk, 2, Q)
            vol = (a[:, 0, :, 0] + a[:, 0, :, 1] + a[:, 1, :, 0] + a[:, 1, :, 1]) * 0.25

        sx = float((1 << k) * (wk - 1)) / 512.0
        sy = float((1 << k) * (hk - 1)) / 256.0
        pbx = (jj + fx) * (float(wk - 1) / 512.0)
        pby = (ii + fy) * (float(hk - 1) / 256.0)
        xs, wx3 = _weight_vectors(pbx, sx, wk, Q)
        ys, wy3 = _weight_vectors(pby, sy, hk, Q)

        masks = [(ys == y).astype(jnp.float32)[None, :] for y in range(hk)]
        rows = []
        for dy in range(3):
            acc = jnp.zeros((wk, Q), jnp.float32)
            for y in range(dy, hk - 2 + dy):
                acc = acc + vol[y] * masks[y - dy]
            rows.append(acc)

        lxi = lax.broadcasted_iota(jnp.int32, (wk, Q), 0)
        win = []
        for dy in range(3):
            row_dy = []
            for dx in range(3):
                m = (lxi == (xs + dx)[None, :]).astype(jnp.float32)
                row_dy.append(jnp.sum(rows[dy] * m, axis=0))
            win.append(row_dy)

        t = [[None] * 9 for _ in range(3)]
        for dy in range(3):
            for xo in range(9):
                t[dy][xo] = (wx3[xo, 0] * win[dy][0]
                             + wx3[xo, 1] * win[dy][1]
                             + wx3[xo, 2] * win[dy][2])

        lvl = []
        for (xo, yo) in _OFFS:
            v = (wy3[yo + 4, 0] * t[0][xo + 4]
                 + wy3[yo + 4, 1] * t[1][xo + 4]
                 + wy3[yo + 4, 2] * t[2][xo + 4])
            lvl.append(v)
        outs.append(jnp.stack(lvl, axis=0))

    out_ref[0] = jnp.stack(outs, axis=1)


@jax.jit
def kernel(feat1, feat2, curr_flow):
    f1 = feat1.reshape(B, C, Q)
    f2t = feat2.reshape(B, C, Q).transpose(0, 2, 1)
    flow = curr_flow.reshape(2, Q)

    out = pl.pallas_call(
        _lookup_body,
        grid=(B,),
        in_specs=[
            pl.BlockSpec((1, Q, C), lambda b: (b, 0, 0)),
            pl.BlockSpec((1, C, Q), lambda b: (b, 0, 0)),
            pl.BlockSpec((2, Q), lambda b: (0, 0)),
        ],
        out_specs=pl.BlockSpec((1, L, 4, Q), lambda b: (b, 0, 0, 0)),
        out_shape=jax.ShapeDtypeStruct((B, L, 4, Q), jnp.float32),
    )(f2t, f1, flow)
    return out.reshape(B, L, 4, H8, W8)
